# baseline (speedup 1.0000x reference)
import functools

import jax
import jax.numpy as jnp
from jax import lax
from jax.experimental import pallas as pl
from jax.experimental.pallas import tpu as pltpu
from jax.experimental.pallas import tpu_sc as plsc

N_NODES = 10000
D = 128
MAX_DEG = 32
BATCH = 1024
F1 = 25
F2 = 10
N_CLASSES = 32

NC = 2
NS = 16
NW = NC * NS
L = 16


def _tc_precompute(feats, W_x1, b_x1, W_n1, b_n1):
    def body(f, wx, bx, wn, bn, gx_out, q_out):
        x = f[...]
        gx_out[...] = jnp.maximum(
            jnp.dot(x, wx[...], preferred_element_type=jnp.float32) + bx[...], 0.0)
        q_out[...] = jnp.dot(x, wn[...], preferred_element_type=jnp.float32) + bn[...]

    return pl.pallas_call(
        body,
        out_shape=(jax.ShapeDtypeStruct((N_NODES, D), jnp.float32),
                   jax.ShapeDtypeStruct((N_NODES, D), jnp.float32)),
    )(feats, W_x1, b_x1.reshape(1, D), W_n1, b_n1.reshape(1, D))


_K2_NPT = 320
_K2_C = 8
_K2_NCH = 40
_K2_DEPTH = 4


def _sc_layer1_table(adj10, q):
    mesh = plsc.VectorSubcoreMesh(core_axis_name="c", subcore_axis_name="s")

    @functools.partial(
        pl.kernel,
        out_type=jax.ShapeDtypeStruct((N_NODES, D), jnp.float32),
        mesh=mesh,
        scratch_types=[
            pltpu.VMEM((_K2_NPT * F2,), jnp.int32),
            [pltpu.VMEM((_K2_C * F2, D), jnp.float32)] * _K2_DEPTH,
            [pltpu.VMEM((_K2_C, D), jnp.float32)] * _K2_DEPTH,
            [pltpu.SemaphoreType.DMA] * _K2_DEPTH,
            [pltpu.SemaphoreType.DMA] * _K2_DEPTH,
        ],
    )
    def k2(adj10_hbm, q_hbm, gn_hbm, idxall, rows, outs, gsems, ssems):
        wid = lax.axis_index("s") * NC + lax.axis_index("c")
        base = jnp.minimum(wid * _K2_NPT, N_NODES - _K2_NPT)
        pltpu.sync_copy(adj10_hbm.at[pl.ds(base * F2, _K2_NPT * F2)], idxall)

        def issue(ch, k):
            pltpu.async_copy(
                q_hbm.at[idxall.at[pl.ds(ch * _K2_C * F2, _K2_C * F2)]],
                rows[k], gsems[k])

        def reduce(rows_k, out_k):
            for i in range(_K2_C):
                for lb in range(D // L):
                    sl = slice(lb * L, (lb + 1) * L)
                    acc = rows_k[i * F2, sl]
                    for j in range(1, F2):
                        acc = acc + rows_k[i * F2 + j, sl]
                    out_k[i, sl] = jnp.maximum(acc * (1.0 / F2), 0.0)

        for k in range(_K2_DEPTH - 1):
            issue(k, k)

        def step(t, carry):
            for k in range(_K2_DEPTH):
                ch = _K2_DEPTH * t + k
                pltpu.make_async_copy(q_hbm.at[pl.ds(0, _K2_C * F2)],
                                      rows[k], gsems[k]).wait()

                @pl.when(t > 0)
                def _():
                    pltpu.make_async_copy(
                        outs[k], gn_hbm.at[pl.ds(0, _K2_C)], ssems[k]).wait()

                reduce(rows[k], outs[k])
                pltpu.async_copy(outs[k],
                                 gn_hbm.at[pl.ds(base + ch * _K2_C, _K2_C)],
                                 ssems[k])
                nxt = ch + _K2_DEPTH - 1

                @pl.when(nxt < _K2_NCH)
                def _():
                    issue(nxt, (k + _K2_DEPTH - 1) % _K2_DEPTH)
            return carry

        lax.fori_loop(0, _K2_NCH // _K2_DEPTH, step, 0)
        for k in range(_K2_DEPTH):
            pltpu.make_async_copy(outs[k], gn_hbm.at[pl.ds(0, _K2_C)],
                                  ssems[k]).wait()

    return k2(adj10, q)


_K3_SPW = BATCH // NW
_K3_C = 2
_K3_NCH = _K3_SPW // _K3_C
_K3_R = _K3_C * MAX_DEG


def _sc_seed_aggregate(ids, adj, q, gx, gn):
    mesh = plsc.VectorSubcoreMesh(core_axis_name="c", subcore_axis_name="s")
    S = jax.ShapeDtypeStruct((BATCH, D), jnp.float32)

    @functools.partial(
        pl.kernel,
        out_type=(S, S, S, S),
        mesh=mesh,
        scratch_types=[
            pltpu.VMEM((_K3_SPW,), jnp.int32),
            pltpu.VMEM((_K3_SPW, 128), jnp.int32),
            pltpu.VMEM((_K3_R,), jnp.int32),
            pltpu.VMEM((_K3_R,), jnp.int32),
            pltpu.VMEM((_K3_R, D), jnp.float32),
            pltpu.VMEM((_K3_R, D), jnp.float32),
            pltpu.VMEM((_K3_R, D), jnp.float32),
            pltpu.VMEM((_K3_R, D), jnp.float32),
            pltpu.VMEM((_K3_R, D), jnp.float32),
            pltpu.VMEM((_K3_R, D), jnp.float32),
            pltpu.VMEM((_K3_SPW, D), jnp.float32),
            pltpu.VMEM((_K3_SPW, D), jnp.float32),
            pltpu.VMEM((_K3_SPW, D), jnp.float32),
            pltpu.VMEM((_K3_SPW, D), jnp.float32),
            pltpu.SemaphoreType.DMA,
            pltpu.SemaphoreType.DMA,
        ],
    )
    def k3(ids_hbm, adj_hbm, q_hbm, gx_hbm, gn_hbm,
           g0top_hbm, g0bot_hbm, mgx_hbm, mgn_hbm,
           sbuf, adjbuf, idxa, idxb, qra, gxra, gnra, qrb, gxrb, gnrb,
           botbuf, mgxbuf, mgnbuf, topbuf, sema, semb):
        wid = lax.axis_index("s") * NC + lax.axis_index("c")
        base = wid * _K3_SPW

        pltpu.sync_copy(ids_hbm.at[pl.ds(base, _K3_SPW)], sbuf)
        pltpu.async_copy(adj_hbm.at[sbuf], adjbuf, sema).wait()
        pltpu.async_copy(gx_hbm.at[sbuf], topbuf, sema).wait()
        pltpu.sync_copy(topbuf, g0top_hbm.at[pl.ds(base, _K3_SPW)])

        def prep_issue(ch, idx, qr, gxr, gnr, sem):
            s0 = ch * _K3_C
            for i in range(_K3_C):
                idx[pl.ds(i * MAX_DEG, L)] = adjbuf[s0 + i, 0:L]
                idx[pl.ds(i * MAX_DEG + L, L)] = adjbuf[s0 + i, L:2 * L]
            for tab, dst in ((q_hbm, qr), (gx_hbm, gxr), (gn_hbm, gnr)):
                for i in range(_K3_C):
                    o = i * MAX_DEG
                    pltpu.async_copy(tab.at[idx.at[pl.ds(o, MAX_DEG)]],
                                     dst.at[pl.ds(o, MAX_DEG)], sem)

        def drain(qr, gxr, gnr, sem):
            for dst in (qr, gxr, gnr):
                for i in range(_K3_C):
                    o = i * MAX_DEG
                    pltpu.make_async_copy(q_hbm.at[pl.ds(0, MAX_DEG)],
                                          dst.at[pl.ds(o, MAX_DEG)],
                                          sem).wait()

        def reduce_from(rows, dstbuf, s0, relu):
            for i in range(_K3_C):
                for lb in range(D // L):
                    sl = slice(lb * L, (lb + 1) * L)
                    acc = rows[i * MAX_DEG, sl]
                    for j in range(1, F1):
                        acc = acc + rows[i * MAX_DEG + j, sl]
                    acc = acc * (1.0 / F1)
                    if relu:
                        acc = jnp.maximum(acc, 0.0)
                    dstbuf[s0 + i, sl] = acc

        def reduce_all(ch, qr, gxr, gnr):
            s0 = ch * _K3_C
            reduce_from(qr, botbuf, s0, relu=True)
            reduce_from(gxr, mgxbuf, s0, relu=False)
            reduce_from(gnr, mgnbuf, s0, relu=False)

        prep_issue(0, idxa, qra, gxra, gnra, sema)

        def pair(t, carry):
            ch0 = 2 * t
            prep_issue(ch0 + 1, idxb, qrb, gxrb, gnrb, semb)
            drain(qra, gxra, gnra, sema)
            reduce_all(ch0, qra, gxra, gnra)

            @pl.when(t < _K3_NCH // 2 - 1)
            def _():
                prep_issue(ch0 + 2, idxa, qra, gxra, gnra, sema)

            drain(qrb, gxrb, gnrb, semb)
            reduce_all(ch0 + 1, qrb, gxrb, gnrb)
            return carry

        lax.fori_loop(0, _K3_NCH // 2, pair, 0)
        pltpu.sync_copy(botbuf, g0bot_hbm.at[pl.ds(base, _K3_SPW)])
        pltpu.sync_copy(mgxbuf, mgx_hbm.at[pl.ds(base, _K3_SPW)])
        pltpu.sync_copy(mgnbuf, mgn_hbm.at[pl.ds(base, _K3_SPW)])

    return k3(ids, adj, q, gx, gn)


def _tc_head(g0top, g0bot, mgx, mgn, W_x2, b_x2, W_n2, b_n2, fc_W, fc_b):
    def body(t, b, mx, mn, wx, bx, wn, bn, fw, fb, out):
        dot = functools.partial(jnp.dot, preferred_element_type=jnp.float32)
        A = jnp.maximum(dot(t[...], wx[0:D, :]) + dot(b[...], wx[D:2 * D, :])
                        + bx[...], 0.0)
        B = jnp.maximum(dot(mx[...], wn[0:D, :]) + dot(mn[...], wn[D:2 * D, :])
                        + bn[...], 0.0)
        nrm = jnp.sqrt(jnp.sum(A * A, axis=1, keepdims=True)
                       + jnp.sum(B * B, axis=1, keepdims=True))
        nrm = jnp.maximum(nrm, 1e-12)
        out[...] = (dot(A, fw[0:D, :]) + dot(B, fw[D:2 * D, :])) / nrm + fb[...]

    return pl.pallas_call(
        body,
        out_shape=jax.ShapeDtypeStruct((BATCH, N_CLASSES), jnp.float32),
    )(g0top, g0bot, mgx, mgn, W_x2, b_x2.reshape(1, D), W_n2,
      b_n2.reshape(1, D), fc_W, fc_b.reshape(1, N_CLASSES))


def kernel(ids, feats, adj, W_x1, b_x1, W_n1, b_n1, W_x2, b_x2, W_n2, b_n2,
           fc_W, fc_b):
    gx, q = _tc_precompute(feats, W_x1, b_x1, W_n1, b_n1)
    adj10 = adj[:, :F2].reshape(-1)
    gn = _sc_layer1_table(adj10, q)
    adj_pad = jnp.pad(adj, ((0, 0), (0, 128 - MAX_DEG)))
    g0top, g0bot, mgx, mgn = _sc_seed_aggregate(ids, adj_pad, q, gx, gn)
    return _tc_head(g0top, g0bot, mgx, mgn, W_x2, b_x2, W_n2, b_n2, fc_W, fc_b)


# speedup vs baseline: 1.3791x; 1.3791x over previous
import functools

import jax
import jax.numpy as jnp
from jax import lax
from jax.experimental import pallas as pl
from jax.experimental.pallas import tpu as pltpu
from jax.experimental.pallas import tpu_sc as plsc

N_NODES = 10000
D = 128
MAX_DEG = 32
BATCH = 1024
F1 = 25
F2 = 10
N_CLASSES = 32

NC = 2
NS = 16
NW = NC * NS
L = 16


def _tc_precompute(feats, W_x1, b_x1, W_n1, b_n1):
    def body(f, wx, bx, wn, bn, gx_out, q_out, t1_out):
        x = f[...]
        gx = jnp.maximum(
            jnp.dot(x, wx[...], preferred_element_type=jnp.float32) + bx[...], 0.0)
        q = jnp.dot(x, wn[...], preferred_element_type=jnp.float32) + bn[...]
        gx_out[...] = gx
        q_out[...] = q
        qb = jax.lax.bitcast_convert_type(
            q.astype(jnp.bfloat16), jnp.uint16).astype(jnp.uint32)
        gb = jax.lax.bitcast_convert_type(
            gx.astype(jnp.bfloat16), jnp.uint16).astype(jnp.uint32)
        t1_out[...] = jax.lax.bitcast_convert_type(qb | (gb << 16), jnp.int32)

    return pl.pallas_call(
        body,
        out_shape=(jax.ShapeDtypeStruct((N_NODES, D), jnp.float32),
                   jax.ShapeDtypeStruct((N_NODES, D), jnp.float32),
                   jax.ShapeDtypeStruct((N_NODES, D), jnp.int32)),
    )(feats, W_x1, b_x1.reshape(1, D), W_n1, b_n1.reshape(1, D))


_K2_NPT = 320
_K2_C = 8
_K2_NCH = 40
_K2_DEPTH = 4


def _sc_layer1_table(adj10, q):
    mesh = plsc.VectorSubcoreMesh(core_axis_name="c", subcore_axis_name="s")

    @functools.partial(
        pl.kernel,
        out_type=jax.ShapeDtypeStruct((N_NODES, D), jnp.int32),
        mesh=mesh,
        scratch_types=[
            pltpu.VMEM((_K2_NPT * F2,), jnp.int32),
            [pltpu.VMEM((_K2_C * F2, D), jnp.float32)] * _K2_DEPTH,
            [pltpu.VMEM((_K2_C, D), jnp.int32)] * _K2_DEPTH,
            [pltpu.SemaphoreType.DMA] * _K2_DEPTH,
            [pltpu.SemaphoreType.DMA] * _K2_DEPTH,
        ],
    )
    def k2(adj10_hbm, q_hbm, gn_hbm, idxall, rows, outs, gsems, ssems):
        wid = lax.axis_index("s") * NC + lax.axis_index("c")
        base = jnp.minimum(wid * _K2_NPT, N_NODES - _K2_NPT)
        pltpu.sync_copy(adj10_hbm.at[pl.ds(base * F2, _K2_NPT * F2)], idxall)

        def issue(ch, k):
            pltpu.async_copy(
                q_hbm.at[idxall.at[pl.ds(ch * _K2_C * F2, _K2_C * F2)]],
                rows[k], gsems[k])

        def reduce(rows_k, out_k):
            for i in range(_K2_C):
                for lb in range(D // L):
                    sl = slice(lb * L, (lb + 1) * L)
                    acc = rows_k[i * F2, sl]
                    for j in range(1, F2):
                        acc = acc + rows_k[i * F2 + j, sl]
                    gn = jnp.maximum(acc * (1.0 / F2), 0.0)
                    u = lax.bitcast_convert_type(gn, jnp.int32)
                    u = u + 0x7FFF + (lax.shift_right_logical(u, 16) & 1)
                    r = lax.shift_right_logical(u, 16)
                    out_k[i, sl] = r | lax.shift_left(r, 16)

        for k in range(_K2_DEPTH - 1):
            issue(k, k)

        def step(t, carry):
            for k in range(_K2_DEPTH):
                ch = _K2_DEPTH * t + k
                pltpu.make_async_copy(q_hbm.at[pl.ds(0, _K2_C * F2)],
                                      rows[k], gsems[k]).wait()

                @pl.when(t > 0)
                def _():
                    pltpu.make_async_copy(
                        outs[k], gn_hbm.at[pl.ds(0, _K2_C)], ssems[k]).wait()

                reduce(rows[k], outs[k])
                pltpu.async_copy(outs[k],
                                 gn_hbm.at[pl.ds(base + ch * _K2_C, _K2_C)],
                                 ssems[k])
                nxt = ch + _K2_DEPTH - 1

                @pl.when(nxt < _K2_NCH)
                def _():
                    issue(nxt, (k + _K2_DEPTH - 1) % _K2_DEPTH)
            return carry

        lax.fori_loop(0, _K2_NCH // _K2_DEPTH, step, 0)
        for k in range(_K2_DEPTH):
            pltpu.make_async_copy(outs[k], gn_hbm.at[pl.ds(0, _K2_C)],
                                  ssems[k]).wait()

    return k2(adj10, q)


_K3_SPW = BATCH // NW
_K3_C = 2
_K3_NCH = _K3_SPW // _K3_C
_K3_R = _K3_C * MAX_DEG


def _sc_seed_aggregate(ids, adj, t1, t2, gx):
    mesh = plsc.VectorSubcoreMesh(core_axis_name="c", subcore_axis_name="s")
    S = jax.ShapeDtypeStruct((BATCH, D), jnp.float32)

    @functools.partial(
        pl.kernel,
        out_type=(S, S, S, S),
        mesh=mesh,
        scratch_types=[
            pltpu.VMEM((_K3_SPW,), jnp.int32),
            pltpu.VMEM((_K3_SPW, 128), jnp.int32),
            pltpu.VMEM((_K3_R,), jnp.int32),
            pltpu.VMEM((_K3_R,), jnp.int32),
            pltpu.VMEM((_K3_R, D), jnp.int32),
            pltpu.VMEM((_K3_R, D), jnp.int32),
            pltpu.VMEM((_K3_R, D), jnp.int32),
            pltpu.VMEM((_K3_R, D), jnp.int32),
            pltpu.VMEM((_K3_SPW, D), jnp.float32),
            pltpu.VMEM((_K3_SPW, D), jnp.float32),
            pltpu.VMEM((_K3_SPW, D), jnp.float32),
            pltpu.VMEM((_K3_SPW, D), jnp.float32),
            pltpu.SemaphoreType.DMA,
            pltpu.SemaphoreType.DMA,
        ],
    )
    def k3(ids_hbm, adj_hbm, t1_hbm, t2_hbm, gx_hbm,
           g0top_hbm, g0bot_hbm, mgx_hbm, mgn_hbm,
           sbuf, adjbuf, idxa, idxb, t1ra, t2ra, t1rb, t2rb,
           botbuf, mgxbuf, mgnbuf, topbuf, sema, semb):
        wid = lax.axis_index("s") * NC + lax.axis_index("c")
        base = wid * _K3_SPW

        pltpu.sync_copy(ids_hbm.at[pl.ds(base, _K3_SPW)], sbuf)
        pltpu.async_copy(adj_hbm.at[sbuf], adjbuf, sema).wait()
        pltpu.async_copy(gx_hbm.at[sbuf], topbuf, sema).wait()
        pltpu.sync_copy(topbuf, g0top_hbm.at[pl.ds(base, _K3_SPW)])

        def prep_issue(ch, idx, t1r, t2r, sem):
            s0 = ch * _K3_C
            for i in range(_K3_C):
                idx[pl.ds(i * MAX_DEG, L)] = adjbuf[s0 + i, 0:L]
                idx[pl.ds(i * MAX_DEG + L, L)] = adjbuf[s0 + i, L:2 * L]
            for tab, dst in ((t1_hbm, t1r), (t2_hbm, t2r)):
                for i in range(_K3_C):
                    o = i * MAX_DEG
                    pltpu.async_copy(tab.at[idx.at[pl.ds(o, MAX_DEG)]],
                                     dst.at[pl.ds(o, MAX_DEG)], sem)

        def drain(t1r, t2r, sem):
            for dst in (t1r, t2r):
                for i in range(_K3_C):
                    o = i * MAX_DEG
                    pltpu.make_async_copy(t1_hbm.at[pl.ds(0, MAX_DEG)],
                                          dst.at[pl.ds(o, MAX_DEG)],
                                          sem).wait()

        def unpack16(rows, r, sl):
            w = rows[r, sl]
            lo = lax.bitcast_convert_type(lax.shift_left(w, 16), jnp.float32)
            hi = lax.bitcast_convert_type(w & jnp.int32(-65536), jnp.float32)
            return lo, hi

        def reduce_all(ch, t1r, t2r):
            s0 = ch * _K3_C
            for i in range(_K3_C):
                for lb in range(D // L):
                    sl = slice(lb * L, (lb + 1) * L)
                    aq, agx = unpack16(t1r, i * MAX_DEG, sl)
                    agn, _ = unpack16(t2r, i * MAX_DEG, sl)
                    for j in range(1, F1):
                        q_j, gx_j = unpack16(t1r, i * MAX_DEG + j, sl)
                        gn_j, _ = unpack16(t2r, i * MAX_DEG + j, sl)
                        aq = aq + q_j
                        agx = agx + gx_j
                        agn = agn + gn_j
                    botbuf[s0 + i, sl] = jnp.maximum(aq * (1.0 / F1), 0.0)
                    mgxbuf[s0 + i, sl] = agx * (1.0 / F1)
                    mgnbuf[s0 + i, sl] = agn * (1.0 / F1)

        prep_issue(0, idxa, t1ra, t2ra, sema)

        def pair(t, carry):
            ch0 = 2 * t
            prep_issue(ch0 + 1, idxb, t1rb, t2rb, semb)
            drain(t1ra, t2ra, sema)
            reduce_all(ch0, t1ra, t2ra)

            @pl.when(t < _K3_NCH // 2 - 1)
            def _():
                prep_issue(ch0 + 2, idxa, t1ra, t2ra, sema)

            drain(t1rb, t2rb, semb)
            reduce_all(ch0 + 1, t1rb, t2rb)
            return carry

        lax.fori_loop(0, _K3_NCH // 2, pair, 0)
        pltpu.sync_copy(botbuf, g0bot_hbm.at[pl.ds(base, _K3_SPW)])
        pltpu.sync_copy(mgxbuf, mgx_hbm.at[pl.ds(base, _K3_SPW)])
        pltpu.sync_copy(mgnbuf, mgn_hbm.at[pl.ds(base, _K3_SPW)])

    return k3(ids, adj, t1, t2, gx)


def _tc_head(g0top, g0bot, mgx, mgn, W_x2, b_x2, W_n2, b_n2, fc_W, fc_b):
    def body(t, b, mx, mn, wx, bx, wn, bn, fw, fb, out):
        dot = functools.partial(jnp.dot, preferred_element_type=jnp.float32)
        A = jnp.maximum(dot(t[...], wx[0:D, :]) + dot(b[...], wx[D:2 * D, :])
                        + bx[...], 0.0)
        B = jnp.maximum(dot(mx[...], wn[0:D, :]) + dot(mn[...], wn[D:2 * D, :])
                        + bn[...], 0.0)
        nrm = jnp.sqrt(jnp.sum(A * A, axis=1, keepdims=True)
                       + jnp.sum(B * B, axis=1, keepdims=True))
        nrm = jnp.maximum(nrm, 1e-12)
        out[...] = (dot(A, fw[0:D, :]) + dot(B, fw[D:2 * D, :])) / nrm + fb[...]

    return pl.pallas_call(
        body,
        out_shape=jax.ShapeDtypeStruct((BATCH, N_CLASSES), jnp.float32),
    )(g0top, g0bot, mgx, mgn, W_x2, b_x2.reshape(1, D), W_n2,
      b_n2.reshape(1, D), fc_W, fc_b.reshape(1, N_CLASSES))


def kernel(ids, feats, adj, W_x1, b_x1, W_n1, b_n1, W_x2, b_x2, W_n2, b_n2,
           fc_W, fc_b):
    gx, q, t1 = _tc_precompute(feats, W_x1, b_x1, W_n1, b_n1)
    adj10 = adj[:, :F2].reshape(-1)
    t2 = _sc_layer1_table(adj10, q)
    adj_pad = jnp.pad(adj, ((0, 0), (0, 128 - MAX_DEG)))
    g0top, g0bot, mgx, mgn = _sc_seed_aggregate(ids, adj_pad, t1, t2, gx)
    return _tc_head(g0top, g0bot, mgx, mgn, W_x2, b_x2, W_n2, b_n2, fc_W, fc_b)


# speedup vs baseline: 1.3988x; 1.0143x over previous
import functools

import jax
import jax.numpy as jnp
from jax import lax
from jax.experimental import pallas as pl
from jax.experimental.pallas import tpu as pltpu
from jax.experimental.pallas import tpu_sc as plsc

N_NODES = 10000
D = 128
MAX_DEG = 32
BATCH = 1024
F1 = 25
F2 = 10
N_CLASSES = 32

NC = 2
NS = 16
NW = NC * NS
L = 16


def _tc_precompute(feats, W_x1, b_x1, W_n1, b_n1):
    def body(f, wx, bx, wn, bn, gx_out, q_out, t1_out):
        x = f[...]
        gx = jnp.maximum(
            jnp.dot(x, wx[...], preferred_element_type=jnp.float32) + bx[...], 0.0)
        q = jnp.dot(x, wn[...], preferred_element_type=jnp.float32) + bn[...]
        gx_out[...] = gx
        q_out[...] = q
        qb = jax.lax.bitcast_convert_type(
            q.astype(jnp.bfloat16), jnp.uint16).astype(jnp.uint32)
        gb = jax.lax.bitcast_convert_type(
            gx.astype(jnp.bfloat16), jnp.uint16).astype(jnp.uint32)
        t1_out[...] = jax.lax.bitcast_convert_type(qb | (gb << 16), jnp.int32)

    return pl.pallas_call(
        body,
        out_shape=(jax.ShapeDtypeStruct((N_NODES, D), jnp.float32),
                   jax.ShapeDtypeStruct((N_NODES, D), jnp.float32),
                   jax.ShapeDtypeStruct((N_NODES, D), jnp.int32)),
    )(feats, W_x1, b_x1.reshape(1, D), W_n1, b_n1.reshape(1, D))


_K2_NPT = 320
_K2_C = 8
_K2_NCH = 40
_K2_DEPTH = 4


def _sc_layer1_table(adj10, q):
    mesh = plsc.VectorSubcoreMesh(core_axis_name="c", subcore_axis_name="s")

    @functools.partial(
        pl.kernel,
        out_type=jax.ShapeDtypeStruct((N_NODES, D), jnp.int32),
        mesh=mesh,
        scratch_types=[
            pltpu.VMEM((_K2_NPT * F2,), jnp.int32),
            [pltpu.VMEM((_K2_C * F2, D), jnp.float32)] * _K2_DEPTH,
            [pltpu.VMEM((_K2_C, D), jnp.int32)] * _K2_DEPTH,
            [pltpu.SemaphoreType.DMA] * _K2_DEPTH,
            [pltpu.SemaphoreType.DMA] * _K2_DEPTH,
        ],
    )
    def k2(adj10_hbm, q_hbm, gn_hbm, idxall, rows, outs, gsems, ssems):
        wid = lax.axis_index("s") * NC + lax.axis_index("c")
        base = jnp.minimum(wid * _K2_NPT, N_NODES - _K2_NPT)
        pltpu.sync_copy(adj10_hbm.at[pl.ds(base * F2, _K2_NPT * F2)], idxall)

        def issue(ch, k):
            h = _K2_C * F2 // 2
            o = ch * _K2_C * F2
            pltpu.async_copy(q_hbm.at[idxall.at[pl.ds(o, h)]],
                             rows[k].at[pl.ds(0, h)], gsems[k])
            pltpu.async_copy(q_hbm.at[idxall.at[pl.ds(o + h, h)]],
                             rows[k].at[pl.ds(h, h)], gsems[k])

        def reduce(rows_k, out_k):
            for i in range(_K2_C):
                for lb in range(D // L):
                    sl = slice(lb * L, (lb + 1) * L)
                    acc = rows_k[i * F2, sl]
                    for j in range(1, F2):
                        acc = acc + rows_k[i * F2 + j, sl]
                    gn = jnp.maximum(acc * (1.0 / F2), 0.0)
                    u = lax.bitcast_convert_type(gn, jnp.int32)
                    u = u + 0x7FFF + (lax.shift_right_logical(u, 16) & 1)
                    r = lax.shift_right_logical(u, 16)
                    out_k[i, sl] = r | lax.shift_left(r, 16)

        for k in range(_K2_DEPTH - 1):
            issue(k, k)

        def step(t, carry):
            for k in range(_K2_DEPTH):
                ch = _K2_DEPTH * t + k
                pltpu.make_async_copy(q_hbm.at[pl.ds(0, _K2_C * F2)],
                                      rows[k], gsems[k]).wait()

                @pl.when(t > 0)
                def _():
                    pltpu.make_async_copy(
                        outs[k], gn_hbm.at[pl.ds(0, _K2_C)], ssems[k]).wait()

                reduce(rows[k], outs[k])
                pltpu.async_copy(outs[k],
                                 gn_hbm.at[pl.ds(base + ch * _K2_C, _K2_C)],
                                 ssems[k])
                nxt = ch + _K2_DEPTH - 1

                @pl.when(nxt < _K2_NCH)
                def _():
                    issue(nxt, (k + _K2_DEPTH - 1) % _K2_DEPTH)
            return carry

        lax.fori_loop(0, _K2_NCH // _K2_DEPTH, step, 0)
        for k in range(_K2_DEPTH):
            pltpu.make_async_copy(outs[k], gn_hbm.at[pl.ds(0, _K2_C)],
                                  ssems[k]).wait()

    return k2(adj10, q)


_K3_SPW = BATCH // NW
_K3_C = 2
_K3_NCH = _K3_SPW // _K3_C
_K3_R = _K3_C * MAX_DEG


def _sc_seed_aggregate(ids, adj, t1, t2, gx):
    mesh = plsc.VectorSubcoreMesh(core_axis_name="c", subcore_axis_name="s")
    S = jax.ShapeDtypeStruct((BATCH, D), jnp.float32)

    @functools.partial(
        pl.kernel,
        out_type=(S, S, S, S),
        mesh=mesh,
        scratch_types=[
            pltpu.VMEM((_K3_SPW,), jnp.int32),
            pltpu.VMEM((_K3_SPW, 128), jnp.int32),
            pltpu.VMEM((_K3_R,), jnp.int32),
            pltpu.VMEM((_K3_R,), jnp.int32),
            pltpu.VMEM((_K3_R, D), jnp.int32),
            pltpu.VMEM((_K3_R, D), jnp.int32),
            pltpu.VMEM((_K3_R, D), jnp.int32),
            pltpu.VMEM((_K3_R, D), jnp.int32),
            pltpu.VMEM((_K3_SPW, D), jnp.float32),
            pltpu.VMEM((_K3_SPW, D), jnp.float32),
            pltpu.VMEM((_K3_SPW, D), jnp.float32),
            pltpu.VMEM((_K3_SPW, D), jnp.float32),
            pltpu.SemaphoreType.DMA,
            pltpu.SemaphoreType.DMA,
        ],
    )
    def k3(ids_hbm, adj_hbm, t1_hbm, t2_hbm, gx_hbm,
           g0top_hbm, g0bot_hbm, mgx_hbm, mgn_hbm,
           sbuf, adjbuf, idxa, idxb, t1ra, t2ra, t1rb, t2rb,
           botbuf, mgxbuf, mgnbuf, topbuf, sema, semb):
        wid = lax.axis_index("s") * NC + lax.axis_index("c")
        base = wid * _K3_SPW

        pltpu.sync_copy(ids_hbm.at[pl.ds(base, _K3_SPW)], sbuf)
        pltpu.async_copy(adj_hbm.at[sbuf], adjbuf, sema).wait()
        pltpu.async_copy(gx_hbm.at[sbuf], topbuf, sema).wait()
        pltpu.sync_copy(topbuf, g0top_hbm.at[pl.ds(base, _K3_SPW)])

        def prep_issue(ch, idx, t1r, t2r, sem):
            s0 = ch * _K3_C
            for i in range(_K3_C):
                idx[pl.ds(i * MAX_DEG, L)] = adjbuf[s0 + i, 0:L]
                idx[pl.ds(i * MAX_DEG + L, L)] = adjbuf[s0 + i, L:2 * L]
            for tab, dst in ((t1_hbm, t1r), (t2_hbm, t2r)):
                for i in range(_K3_C):
                    o = i * MAX_DEG
                    pltpu.async_copy(tab.at[idx.at[pl.ds(o, MAX_DEG)]],
                                     dst.at[pl.ds(o, MAX_DEG)], sem)

        def drain(t1r, t2r, sem):
            for dst in (t1r, t2r):
                for i in range(_K3_C):
                    o = i * MAX_DEG
                    pltpu.make_async_copy(t1_hbm.at[pl.ds(0, MAX_DEG)],
                                          dst.at[pl.ds(o, MAX_DEG)],
                                          sem).wait()

        def unpack16(rows, r, sl):
            w = rows[r, sl]
            lo = lax.bitcast_convert_type(lax.shift_left(w, 16), jnp.float32)
            hi = lax.bitcast_convert_type(w & jnp.int32(-65536), jnp.float32)
            return lo, hi

        def reduce_all(ch, t1r, t2r):
            s0 = ch * _K3_C
            for i in range(_K3_C):
                for lb in range(D // L):
                    sl = slice(lb * L, (lb + 1) * L)
                    aq, agx = unpack16(t1r, i * MAX_DEG, sl)
                    agn, _ = unpack16(t2r, i * MAX_DEG, sl)
                    for j in range(1, F1):
                        q_j, gx_j = unpack16(t1r, i * MAX_DEG + j, sl)
                        gn_j, _ = unpack16(t2r, i * MAX_DEG + j, sl)
                        aq = aq + q_j
                        agx = agx + gx_j
                        agn = agn + gn_j
                    botbuf[s0 + i, sl] = jnp.maximum(aq * (1.0 / F1), 0.0)
                    mgxbuf[s0 + i, sl] = agx * (1.0 / F1)
                    mgnbuf[s0 + i, sl] = agn * (1.0 / F1)

        prep_issue(0, idxa, t1ra, t2ra, sema)

        def pair(t, carry):
            ch0 = 2 * t
            prep_issue(ch0 + 1, idxb, t1rb, t2rb, semb)
            drain(t1ra, t2ra, sema)
            reduce_all(ch0, t1ra, t2ra)

            @pl.when(t < _K3_NCH // 2 - 1)
            def _():
                prep_issue(ch0 + 2, idxa, t1ra, t2ra, sema)

            drain(t1rb, t2rb, semb)
            reduce_all(ch0 + 1, t1rb, t2rb)
            return carry

        lax.fori_loop(0, _K3_NCH // 2, pair, 0)
        pltpu.sync_copy(botbuf, g0bot_hbm.at[pl.ds(base, _K3_SPW)])
        pltpu.sync_copy(mgxbuf, mgx_hbm.at[pl.ds(base, _K3_SPW)])
        pltpu.sync_copy(mgnbuf, mgn_hbm.at[pl.ds(base, _K3_SPW)])

    return k3(ids, adj, t1, t2, gx)


def _tc_head(g0top, g0bot, mgx, mgn, W_x2, b_x2, W_n2, b_n2, fc_W, fc_b):
    def body(t, b, mx, mn, wx, bx, wn, bn, fw, fb, out):
        dot = functools.partial(jnp.dot, preferred_element_type=jnp.float32)
        A = jnp.maximum(dot(t[...], wx[0:D, :]) + dot(b[...], wx[D:2 * D, :])
                        + bx[...], 0.0)
        B = jnp.maximum(dot(mx[...], wn[0:D, :]) + dot(mn[...], wn[D:2 * D, :])
                        + bn[...], 0.0)
        nrm = jnp.sqrt(jnp.sum(A * A, axis=1, keepdims=True)
                       + jnp.sum(B * B, axis=1, keepdims=True))
        nrm = jnp.maximum(nrm, 1e-12)
        out[...] = (dot(A, fw[0:D, :]) + dot(B, fw[D:2 * D, :])) / nrm + fb[...]

    return pl.pallas_call(
        body,
        out_shape=jax.ShapeDtypeStruct((BATCH, N_CLASSES), jnp.float32),
    )(g0top, g0bot, mgx, mgn, W_x2, b_x2.reshape(1, D), W_n2,
      b_n2.reshape(1, D), fc_W, fc_b.reshape(1, N_CLASSES))


def kernel(ids, feats, adj, W_x1, b_x1, W_n1, b_n1, W_x2, b_x2, W_n2, b_n2,
           fc_W, fc_b):
    gx, q, t1 = _tc_precompute(feats, W_x1, b_x1, W_n1, b_n1)
    adj10 = adj[:, :F2].reshape(-1)
    t2 = _sc_layer1_table(adj10, q)
    adj_pad = jnp.pad(adj, ((0, 0), (0, 128 - MAX_DEG)))
    g0top, g0bot, mgx, mgn = _sc_seed_aggregate(ids, adj_pad, t1, t2, gx)
    return _tc_head(g0top, g0bot, mgx, mgn, W_x2, b_x2, W_n2, b_n2, fc_W, fc_b)


# speedup vs baseline: 1.6130x; 1.1531x over previous
import functools

import jax
import jax.numpy as jnp
from jax import lax
from jax.experimental import pallas as pl
from jax.experimental.pallas import tpu as pltpu
from jax.experimental.pallas import tpu_sc as plsc

N_NODES = 10000
D = 128
MAX_DEG = 32
BATCH = 1024
F1 = 25
F2 = 10
N_CLASSES = 32

NC = 2
NS = 16
NW = NC * NS
L = 16


def _tc_precompute(feats, W_x1, b_x1, W_n1, b_n1):
    def body(f, wx, bx, wn, bn, t1_out):
        x = f[...]
        gx = jnp.maximum(
            jnp.dot(x, wx[...], preferred_element_type=jnp.float32) + bx[...], 0.0)
        q = jnp.dot(x, wn[...], preferred_element_type=jnp.float32) + bn[...]
        qb = jax.lax.bitcast_convert_type(
            q.astype(jnp.bfloat16), jnp.uint16).astype(jnp.uint32)
        gb = jax.lax.bitcast_convert_type(
            gx.astype(jnp.bfloat16), jnp.uint16).astype(jnp.uint32)
        t1_out[...] = jax.lax.bitcast_convert_type(qb | (gb << 16), jnp.int32)

    return pl.pallas_call(
        body,
        out_shape=jax.ShapeDtypeStruct((N_NODES, D), jnp.int32),
    )(feats, W_x1, b_x1.reshape(1, D), W_n1, b_n1.reshape(1, D))


_K2_NPT = 320
_K2_C = 8
_K2_NCH = 40
_K2_DEPTH = 4


def _sc_layer1_table(adj10, t1):
    mesh = plsc.VectorSubcoreMesh(core_axis_name="c", subcore_axis_name="s")

    @functools.partial(
        pl.kernel,
        out_type=jax.ShapeDtypeStruct((N_NODES, D), jnp.int32),
        mesh=mesh,
        scratch_types=[
            pltpu.VMEM((_K2_NPT * F2,), jnp.int32),
            [pltpu.VMEM((_K2_C * F2, D), jnp.int32)] * _K2_DEPTH,
            [pltpu.VMEM((_K2_C, D), jnp.int32)] * _K2_DEPTH,
            [pltpu.SemaphoreType.DMA] * _K2_DEPTH,
            [pltpu.SemaphoreType.DMA] * _K2_DEPTH,
        ],
    )
    def k2(adj10_hbm, t1_hbm, gn_hbm, idxall, rows, outs, gsems, ssems):
        wid = lax.axis_index("s") * NC + lax.axis_index("c")
        base = jnp.minimum(wid * _K2_NPT, N_NODES - _K2_NPT)
        pltpu.sync_copy(adj10_hbm.at[pl.ds(base * F2, _K2_NPT * F2)], idxall)

        def issue(ch, k):
            h = _K2_C * F2 // 2
            o = ch * _K2_C * F2
            pltpu.async_copy(t1_hbm.at[idxall.at[pl.ds(o, h)]],
                             rows[k].at[pl.ds(0, h)], gsems[k])
            pltpu.async_copy(t1_hbm.at[idxall.at[pl.ds(o + h, h)]],
                             rows[k].at[pl.ds(h, h)], gsems[k])

        def reduce(rows_k, out_k):
            for i in range(_K2_C):
                for lb in range(D // L):
                    sl = slice(lb * L, (lb + 1) * L)
                    qlo = lambda r: lax.bitcast_convert_type(
                        lax.shift_left(rows_k[r, sl], 16), jnp.float32)
                    acc = qlo(i * F2)
                    for j in range(1, F2):
                        acc = acc + qlo(i * F2 + j)
                    gn = jnp.maximum(acc * (1.0 / F2), 0.0)
                    u = lax.bitcast_convert_type(gn, jnp.int32)
                    u = u + 0x7FFF + (lax.shift_right_logical(u, 16) & 1)
                    r = lax.shift_right_logical(u, 16)
                    out_k[i, sl] = r | lax.shift_left(r, 16)

        for k in range(_K2_DEPTH - 1):
            issue(k, k)

        def step(t, carry):
            for k in range(_K2_DEPTH):
                ch = _K2_DEPTH * t + k
                pltpu.make_async_copy(t1_hbm.at[pl.ds(0, _K2_C * F2)],
                                      rows[k], gsems[k]).wait()

                @pl.when(t > 0)
                def _():
                    pltpu.make_async_copy(
                        outs[k], gn_hbm.at[pl.ds(0, _K2_C)], ssems[k]).wait()

                reduce(rows[k], outs[k])
                pltpu.async_copy(outs[k],
                                 gn_hbm.at[pl.ds(base + ch * _K2_C, _K2_C)],
                                 ssems[k])
                nxt = ch + _K2_DEPTH - 1

                @pl.when(nxt < _K2_NCH)
                def _():
                    issue(nxt, (k + _K2_DEPTH - 1) % _K2_DEPTH)
            return carry

        lax.fori_loop(0, _K2_NCH // _K2_DEPTH, step, 0)
        for k in range(_K2_DEPTH):
            pltpu.make_async_copy(outs[k], gn_hbm.at[pl.ds(0, _K2_C)],
                                  ssems[k]).wait()

    return k2(adj10, t1)


_K3_SPW = BATCH // NW
_K3_C = 2
_K3_NCH = _K3_SPW // _K3_C
_K3_R = _K3_C * MAX_DEG


def _sc_seed_aggregate(ids, adj, t1, t2):
    mesh = plsc.VectorSubcoreMesh(core_axis_name="c", subcore_axis_name="s")
    S = jax.ShapeDtypeStruct((BATCH, D), jnp.float32)

    @functools.partial(
        pl.kernel,
        out_type=(S, S, S, S),
        mesh=mesh,
        scratch_types=[
            pltpu.VMEM((_K3_SPW,), jnp.int32),
            pltpu.VMEM((_K3_SPW, 128), jnp.int32),
            pltpu.VMEM((_K3_R,), jnp.int32),
            pltpu.VMEM((_K3_R,), jnp.int32),
            pltpu.VMEM((_K3_R, D), jnp.int32),
            pltpu.VMEM((_K3_R, D), jnp.int32),
            pltpu.VMEM((_K3_R, D), jnp.int32),
            pltpu.VMEM((_K3_R, D), jnp.int32),
            pltpu.VMEM((_K3_SPW, D), jnp.float32),
            pltpu.VMEM((_K3_SPW, D), jnp.float32),
            pltpu.VMEM((_K3_SPW, D), jnp.float32),
            pltpu.VMEM((_K3_SPW, D), jnp.int32),
            pltpu.SemaphoreType.DMA,
            pltpu.SemaphoreType.DMA,
        ],
    )
    def k3(ids_hbm, adj_hbm, t1_hbm, t2_hbm,
           g0top_hbm, g0bot_hbm, mgx_hbm, mgn_hbm,
           sbuf, adjbuf, idxa, idxb, t1ra, t2ra, t1rb, t2rb,
           botbuf, mgxbuf, mgnbuf, topbuf, sema, semb):
        wid = lax.axis_index("s") * NC + lax.axis_index("c")
        base = wid * _K3_SPW

        pltpu.sync_copy(ids_hbm.at[pl.ds(base, _K3_SPW)], sbuf)
        pltpu.async_copy(adj_hbm.at[sbuf], adjbuf, sema).wait()
        pltpu.async_copy(t1_hbm.at[sbuf], topbuf, sema).wait()
        for i in range(_K3_SPW):
            for lb in range(D // L):
                sl = slice(lb * L, (lb + 1) * L)
                botbuf[i, sl] = lax.bitcast_convert_type(
                    topbuf[i, sl] & jnp.int32(-65536), jnp.float32)
        pltpu.sync_copy(botbuf, g0top_hbm.at[pl.ds(base, _K3_SPW)])

        def prep_issue(ch, idx, t1r, t2r, sem):
            s0 = ch * _K3_C
            for i in range(_K3_C):
                idx[pl.ds(i * MAX_DEG, L)] = adjbuf[s0 + i, 0:L]
                idx[pl.ds(i * MAX_DEG + L, L)] = adjbuf[s0 + i, L:2 * L]
            for tab, dst in ((t1_hbm, t1r), (t2_hbm, t2r)):
                for i in range(_K3_C):
                    o = i * MAX_DEG
                    pltpu.async_copy(tab.at[idx.at[pl.ds(o, MAX_DEG)]],
                                     dst.at[pl.ds(o, MAX_DEG)], sem)

        def drain(t1r, t2r, sem):
            for dst in (t1r, t2r):
                for i in range(_K3_C):
                    o = i * MAX_DEG
                    pltpu.make_async_copy(t1_hbm.at[pl.ds(0, MAX_DEG)],
                                          dst.at[pl.ds(o, MAX_DEG)],
                                          sem).wait()

        def unpack16(rows, r, sl):
            w = rows[r, sl]
            lo = lax.bitcast_convert_type(lax.shift_left(w, 16), jnp.float32)
            hi = lax.bitcast_convert_type(w & jnp.int32(-65536), jnp.float32)
            return lo, hi

        def reduce_all(ch, t1r, t2r):
            s0 = ch * _K3_C
            for i in range(_K3_C):
                for lb in range(D // L):
                    sl = slice(lb * L, (lb + 1) * L)
                    aq, agx = unpack16(t1r, i * MAX_DEG, sl)
                    agn, _ = unpack16(t2r, i * MAX_DEG, sl)
                    for j in range(1, F1):
                        q_j, gx_j = unpack16(t1r, i * MAX_DEG + j, sl)
                        gn_j, _ = unpack16(t2r, i * MAX_DEG + j, sl)
                        aq = aq + q_j
                        agx = agx + gx_j
                        agn = agn + gn_j
                    botbuf[s0 + i, sl] = jnp.maximum(aq * (1.0 / F1), 0.0)
                    mgxbuf[s0 + i, sl] = agx * (1.0 / F1)
                    mgnbuf[s0 + i, sl] = agn * (1.0 / F1)

        prep_issue(0, idxa, t1ra, t2ra, sema)

        def pair(t, carry):
            ch0 = 2 * t
            prep_issue(ch0 + 1, idxb, t1rb, t2rb, semb)
            drain(t1ra, t2ra, sema)
            reduce_all(ch0, t1ra, t2ra)

            @pl.when(t < _K3_NCH // 2 - 1)
            def _():
                prep_issue(ch0 + 2, idxa, t1ra, t2ra, sema)

            drain(t1rb, t2rb, semb)
            reduce_all(ch0 + 1, t1rb, t2rb)
            return carry

        lax.fori_loop(0, _K3_NCH // 2, pair, 0)
        pltpu.sync_copy(botbuf, g0bot_hbm.at[pl.ds(base, _K3_SPW)])
        pltpu.sync_copy(mgxbuf, mgx_hbm.at[pl.ds(base, _K3_SPW)])
        pltpu.sync_copy(mgnbuf, mgn_hbm.at[pl.ds(base, _K3_SPW)])

    return k3(ids, adj, t1, t2)


def _tc_head(g0top, g0bot, mgx, mgn, W_x2, b_x2, W_n2, b_n2, fc_W, fc_b):
    def body(t, b, mx, mn, wx, bx, wn, bn, fw, fb, out):
        dot = functools.partial(jnp.dot, preferred_element_type=jnp.float32)
        A = jnp.maximum(dot(t[...], wx[0:D, :]) + dot(b[...], wx[D:2 * D, :])
                        + bx[...], 0.0)
        B = jnp.maximum(dot(mx[...], wn[0:D, :]) + dot(mn[...], wn[D:2 * D, :])
                        + bn[...], 0.0)
        nrm = jnp.sqrt(jnp.sum(A * A, axis=1, keepdims=True)
                       + jnp.sum(B * B, axis=1, keepdims=True))
        nrm = jnp.maximum(nrm, 1e-12)
        out[...] = (dot(A, fw[0:D, :]) + dot(B, fw[D:2 * D, :])) / nrm + fb[...]

    return pl.pallas_call(
        body,
        out_shape=jax.ShapeDtypeStruct((BATCH, N_CLASSES), jnp.float32),
    )(g0top, g0bot, mgx, mgn, W_x2, b_x2.reshape(1, D), W_n2,
      b_n2.reshape(1, D), fc_W, fc_b.reshape(1, N_CLASSES))


def kernel(ids, feats, adj, W_x1, b_x1, W_n1, b_n1, W_x2, b_x2, W_n2, b_n2,
           fc_W, fc_b):
    t1 = _tc_precompute(feats, W_x1, b_x1, W_n1, b_n1)
    adj10 = adj[:, :F2].reshape(-1)
    t2 = _sc_layer1_table(adj10, t1)
    adj_pad = jnp.pad(adj, ((0, 0), (0, 128 - MAX_DEG)))
    g0top, g0bot, mgx, mgn = _sc_seed_aggregate(ids, adj_pad, t1, t2)
    return _tc_head(g0top, g0bot, mgx, mgn, W_x2, b_x2, W_n2, b_n2, fc_W, fc_b)
